# baseline (speedup 1.0000x reference)
import functools

import jax
import jax.numpy as jnp
from jax import lax
from jax.experimental import pallas as pl
from jax.experimental.pallas import tpu as pltpu
from jax.experimental.pallas import tpu_sc as plsc

N = 10000
E = 320000
D = 128

NC = 2
NS = 16
NW = NC * NS
EPT = E // NW
B = 80
NB = EPT // B
NPAD = 10240
RPT = NPAD // NS

_MESH = plsc.VectorSubcoreMesh(core_axis_name="c", subcore_axis_name="s")


@functools.partial(
    pl.kernel,
    out_type=jax.ShapeDtypeStruct((NW, NPAD), jnp.float32),
    mesh=_MESH,
    scratch_types=[
        pltpu.VMEM((NB, B), jnp.int32),
        pltpu.VMEM((NPAD,), jnp.float32),
    ],
    compiler_params=pltpu.CompilerParams(needs_layout_passes=False),
)
def _sc_degree(dst_hbm, zeros_hbm, out_hbm, dst_v, acc):
    c = lax.axis_index("c")
    s = lax.axis_index("s")
    wid = s * NC + c
    pltpu.sync_copy(zeros_hbm, acc)
    pltpu.sync_copy(dst_hbm.at[wid], dst_v)
    ones16 = jnp.full((16,), 1.0, jnp.float32)

    def body(m, carry):
        for r in range(B // 16):
            d16 = dst_v[m, pl.ds(r * 16, 16)]
            plsc.addupdate_scatter(acc, [d16], ones16)
        return carry

    lax.fori_loop(0, NB, body, 0)
    pltpu.sync_copy(acc, out_hbm.at[wid])


_NRING = 3


NCH = 5
CHB = NB // NCH


@functools.partial(
    pl.kernel,
    out_type=jax.ShapeDtypeStruct((NC, NPAD, D), jnp.float32),
    mesh=_MESH,
    scratch_types=(
        [pltpu.VMEM((CHB, B), jnp.int32) for _ in range(2)]
        + [pltpu.VMEM((B, D), jnp.float32) for _ in range(_NRING)]
        + [pltpu.SemaphoreType.DMA for _ in range(_NRING)]
        + [pltpu.VMEM_SHARED((NPAD, D), jnp.float32)]
    ),
)
def _sc_prop_rows(src_hbm, dst_hbm, tab_hbm, zeros_hbm, out_hbm, *refs):
    src_v, dst_v = refs[0], refs[1]
    bufs = refs[2:2 + _NRING]
    gsem = refs[2 + _NRING:2 + 2 * _NRING]
    acc = refs[2 + 2 * _NRING]
    c = lax.axis_index("c")
    s = lax.axis_index("s")
    wid = s * NC + c
    rows = pl.ds(s * RPT, RPT)
    pltpu.sync_copy(zeros_hbm, acc.at[rows])
    plsc.subcore_barrier()

    dummy = tab_hbm.at[pl.ds(0, B)]

    def wait_g(r):
        pltpu.make_async_copy(dummy, bufs[r], gsem[r]).wait()

    def gath(j, r):
        pltpu.async_copy(tab_hbm.at[src_v.at[j]], bufs[r], gsem[r])

    def scat(j, r):
        pltpu.sync_copy(bufs[r], acc.at[dst_v.at[j]], add=True)

    for ch in range(NCH):
        pltpu.sync_copy(src_hbm.at[wid, ch], src_v)
        pltpu.sync_copy(dst_hbm.at[wid, ch], dst_v)
        gath(0, 0)
        gath(1, 1)

        def triple(m, carry):
            for r in range(3):
                j = 3 * m + r
                gath(j + 2, (r + 2) % 3)
                wait_g(r)
                scat(j, r)
            return carry

        lax.fori_loop(0, (CHB - 4) // 3, triple, 0)
        for j in range(CHB - 4, CHB):
            if j + 2 < CHB:
                gath(j + 2, (j + 2) % 3)
            wait_g(j % 3)
            scat(j, j % 3)
    plsc.subcore_barrier()
    pltpu.sync_copy(acc.at[rows], out_hbm.at[c, rows])


@functools.partial(
    pl.kernel,
    out_type=jax.ShapeDtypeStruct((NW, NPAD), jnp.float32),
    mesh=_MESH,
    scratch_types=[
        pltpu.VMEM((NB, B), jnp.int32),
        pltpu.VMEM((NB, B), jnp.int32),
        pltpu.VMEM((NPAD,), jnp.float32),
        pltpu.VMEM((NPAD,), jnp.float32),
    ],
    compiler_params=pltpu.CompilerParams(needs_layout_passes=False),
)
def _sc_prop_scalar(src_hbm, dst_hbm, tab_hbm, zeros_hbm, out_hbm,
                    src_v, dst_v, tab_v, acc):
    c = lax.axis_index("c")
    s = lax.axis_index("s")
    wid = s * NC + c
    pltpu.sync_copy(zeros_hbm, acc)
    pltpu.sync_copy(tab_hbm, tab_v)
    pltpu.sync_copy(src_hbm.at[wid], src_v)
    pltpu.sync_copy(dst_hbm.at[wid], dst_v)

    def body(m, carry):
        for r in range(B // 16):
            s16 = src_v[m, pl.ds(r * 16, 16)]
            d16 = dst_v[m, pl.ds(r * 16, 16)]
            vals = plsc.load_gather(tab_v, [s16])
            plsc.addupdate_scatter(acc, [d16], vals)
        return carry

    lax.fori_loop(0, NB, body, 0)
    pltpu.sync_copy(acc, out_hbm.at[wid])


_GRID = 20
_BR = NPAD // _GRID


def _col_sum(parts):
    ones = jnp.ones((NW, 1), jnp.float32)
    return lax.dot_general(parts, ones, (((0,), (0,)), ((), ())),
                           preferred_element_type=jnp.float32)


def _tc_mm1(x_ref, w1_ref, h_ref):
    h_ref[...] = jnp.dot(x_ref[...], w1_ref[...],
                         preferred_element_type=jnp.float32)


def _tc_scale(h_ref, degp_ref, hn_ref, dinv_ref):
    deg = _col_sum(degp_ref[...]) + 1.0
    dinv = lax.rsqrt(deg)
    hn_ref[...] = h_ref[...] * dinv
    dinv_ref[...] = dinv


def _tc_mid(accp_ref, hn_ref, dinv_ref, b1_ref, w2_ref, pn_ref):
    ssum = accp_ref[0] + accp_ref[1] + hn_ref[...]
    out1 = jnp.maximum(ssum * dinv_ref[...] + b1_ref[...][None, :], 0.0)
    q = jnp.dot(out1, w2_ref[...], preferred_element_type=jnp.float32)
    pn_ref[...] = q * dinv_ref[...]


def _tc_out(acc2p_ref, pn_ref, dinv_ref, b2_ref, out_ref):
    s2 = _col_sum(acc2p_ref[...]) + pn_ref[...]
    out_ref[...] = jax.nn.sigmoid(s2 * dinv_ref[...] + b2_ref[...])


def kernel(x, edge_index, W1, b1, W2, b2):
    src = edge_index[0].astype(jnp.int32).reshape(NW, NB, B)
    dst = edge_index[1].astype(jnp.int32).reshape(NW, NB, B)
    x_pad = jnp.zeros((NPAD, D), jnp.float32).at[:N].set(x)
    zeros_big = jnp.zeros((RPT, D), jnp.float32)
    zeros_small = jnp.zeros((NPAD,), jnp.float32)

    degp = _sc_degree(dst, zeros_small)

    h = pl.pallas_call(
        _tc_mm1,
        grid=(_GRID,),
        in_specs=[
            pl.BlockSpec((_BR, D), lambda i: (i, 0)),
            pl.BlockSpec((D, D), lambda i: (0, 0)),
        ],
        out_specs=pl.BlockSpec((_BR, D), lambda i: (i, 0)),
        out_shape=jax.ShapeDtypeStruct((NPAD, D), jnp.float32),
    )(x_pad, W1)

    hn, dinv = pl.pallas_call(
        _tc_scale,
        grid=(_GRID,),
        in_specs=[
            pl.BlockSpec((_BR, D), lambda i: (i, 0)),
            pl.BlockSpec((NW, _BR), lambda i: (0, i)),
        ],
        out_specs=[
            pl.BlockSpec((_BR, D), lambda i: (i, 0)),
            pl.BlockSpec((_BR, 1), lambda i: (i, 0)),
        ],
        out_shape=[
            jax.ShapeDtypeStruct((NPAD, D), jnp.float32),
            jax.ShapeDtypeStruct((NPAD, 1), jnp.float32),
        ],
    )(h, degp)

    accp = _sc_prop_rows(src.reshape(NW, NCH, CHB, B),
                         dst.reshape(NW, NCH, CHB, B), hn, zeros_big)

    pn = pl.pallas_call(
        _tc_mid,
        grid=(_GRID,),
        in_specs=[
            pl.BlockSpec((NC, _BR, D), lambda i: (0, i, 0)),
            pl.BlockSpec((_BR, D), lambda i: (i, 0)),
            pl.BlockSpec((_BR, 1), lambda i: (i, 0)),
            pl.BlockSpec((D,), lambda i: (0,)),
            pl.BlockSpec((D, 1), lambda i: (0, 0)),
        ],
        out_specs=pl.BlockSpec((_BR, 1), lambda i: (i, 0)),
        out_shape=jax.ShapeDtypeStruct((NPAD, 1), jnp.float32),
    )(accp, hn, dinv, b1, W2)

    acc2p = _sc_prop_scalar(src, dst, pn.reshape(NPAD), zeros_small)

    out = pl.pallas_call(
        _tc_out,
        grid=(_GRID,),
        in_specs=[
            pl.BlockSpec((NW, _BR), lambda i: (0, i)),
            pl.BlockSpec((_BR, 1), lambda i: (i, 0)),
            pl.BlockSpec((_BR, 1), lambda i: (i, 0)),
            pl.BlockSpec((1,), lambda i: (0,)),
        ],
        out_specs=pl.BlockSpec((_BR, 1), lambda i: (i, 0)),
        out_shape=jax.ShapeDtypeStruct((NPAD, 1), jnp.float32),
    )(acc2p, pn, dinv, b2)

    return out[:N]


# speedup vs baseline: 1.0488x; 1.0488x over previous
import functools

import jax
import jax.numpy as jnp
from jax import lax
from jax.experimental import pallas as pl
from jax.experimental.pallas import tpu as pltpu
from jax.experimental.pallas import tpu_sc as plsc

N = 10000
E = 320000
D = 128

NC = 2
NS = 16
NW = NC * NS
EPT = E // NW
B = 80
NB = EPT // B
NPAD = 10240
RPT = NPAD // NS

_MESH = plsc.VectorSubcoreMesh(core_axis_name="c", subcore_axis_name="s")


@functools.partial(
    pl.kernel,
    out_type=jax.ShapeDtypeStruct((NW, NPAD), jnp.float32),
    mesh=_MESH,
    scratch_types=[
        pltpu.VMEM((NB, B), jnp.int32),
        pltpu.VMEM((NPAD,), jnp.float32),
    ],
    compiler_params=pltpu.CompilerParams(needs_layout_passes=False),
)
def _sc_degree(dst_hbm, zeros_hbm, out_hbm, dst_v, acc):
    c = lax.axis_index("c")
    s = lax.axis_index("s")
    wid = s * NC + c
    pltpu.sync_copy(zeros_hbm, acc)
    pltpu.sync_copy(dst_hbm.at[wid], dst_v)
    ones16 = jnp.full((16,), 1.0, jnp.float32)

    def body(m, carry):
        for r in range(B // 16):
            d16 = dst_v[m, pl.ds(r * 16, 16)]
            plsc.addupdate_scatter(acc, [d16], ones16)
        return carry

    lax.fori_loop(0, NB, body, 0)
    pltpu.sync_copy(acc, out_hbm.at[wid])


_NRING = 3


NCH = 5
CHB = NB // NCH


@functools.partial(
    pl.kernel,
    out_type=jax.ShapeDtypeStruct((NC, NPAD, D), jnp.float32),
    mesh=_MESH,
    scratch_types=(
        [pltpu.VMEM((CHB, B), jnp.int32) for _ in range(2)]
        + [pltpu.VMEM((B, D), jnp.float32) for _ in range(_NRING)]
        + [pltpu.SemaphoreType.DMA for _ in range(2 * _NRING)]
        + [pltpu.VMEM_SHARED((NPAD, D), jnp.float32)]
    ),
)
def _sc_prop_rows(src_hbm, dst_hbm, tab_hbm, zeros_hbm, out_hbm, *refs):
    src_v, dst_v = refs[0], refs[1]
    bufs = refs[2:2 + _NRING]
    gsem = refs[2 + _NRING:2 + 2 * _NRING]
    ssem = refs[2 + 2 * _NRING:2 + 3 * _NRING]
    acc = refs[2 + 3 * _NRING]
    c = lax.axis_index("c")
    s = lax.axis_index("s")
    wid = s * NC + c
    rows = pl.ds(s * RPT, RPT)
    pltpu.sync_copy(zeros_hbm, acc.at[rows])
    plsc.subcore_barrier()

    dummy = tab_hbm.at[pl.ds(0, B)]

    def wait_g(r):
        pltpu.make_async_copy(dummy, bufs[r], gsem[r]).wait()

    def gath(j, r):
        pltpu.async_copy(tab_hbm.at[src_v.at[j]], bufs[r], gsem[r])

    def wait_s(r):
        pltpu.make_async_copy(dummy, bufs[r], ssem[r]).wait()

    def scat(j, r):
        pltpu.async_copy(bufs[r], acc.at[dst_v.at[j]], ssem[r], add=True)

    for ch in range(NCH):
        pltpu.sync_copy(src_hbm.at[wid, ch], src_v)
        pltpu.sync_copy(dst_hbm.at[wid, ch], dst_v)
        gath(0, 0)
        gath(1, 1)
        wait_g(0)
        scat(0, 0)
        gath(2, 2)

        def triple(m, carry):
            for r in range(3):
                j = 3 * m + 1 + r
                rb = (1 + r) % 3
                wait_g(rb)
                scat(j, rb)
                wait_s(r)
                gath(j + 2, r)
            return carry

        lax.fori_loop(0, (CHB - 4) // 3, triple, 0)
        wait_g((CHB - 3) % 3)
        scat(CHB - 3, (CHB - 3) % 3)
        wait_s((CHB - 1) % 3)
        gath(CHB - 1, (CHB - 1) % 3)
        for j in range(CHB - 2, CHB):
            wait_g(j % 3)
            scat(j, j % 3)
        for r in range(3):
            wait_s(r)
    plsc.subcore_barrier()
    pltpu.sync_copy(acc.at[rows], out_hbm.at[c, rows])


@functools.partial(
    pl.kernel,
    out_type=jax.ShapeDtypeStruct((NW, NPAD), jnp.float32),
    mesh=_MESH,
    scratch_types=[
        pltpu.VMEM((NB, B), jnp.int32),
        pltpu.VMEM((NB, B), jnp.int32),
        pltpu.VMEM((NPAD,), jnp.float32),
        pltpu.VMEM((NPAD,), jnp.float32),
    ],
    compiler_params=pltpu.CompilerParams(needs_layout_passes=False),
)
def _sc_prop_scalar(src_hbm, dst_hbm, tab_hbm, zeros_hbm, out_hbm,
                    src_v, dst_v, tab_v, acc):
    c = lax.axis_index("c")
    s = lax.axis_index("s")
    wid = s * NC + c
    pltpu.sync_copy(zeros_hbm, acc)
    pltpu.sync_copy(tab_hbm, tab_v)
    pltpu.sync_copy(src_hbm.at[wid], src_v)
    pltpu.sync_copy(dst_hbm.at[wid], dst_v)

    def body(m, carry):
        for r in range(B // 16):
            s16 = src_v[m, pl.ds(r * 16, 16)]
            d16 = dst_v[m, pl.ds(r * 16, 16)]
            vals = plsc.load_gather(tab_v, [s16])
            plsc.addupdate_scatter(acc, [d16], vals)
        return carry

    lax.fori_loop(0, NB, body, 0)
    pltpu.sync_copy(acc, out_hbm.at[wid])


_GRID = 20
_BR = NPAD // _GRID


def _col_sum(parts):
    ones = jnp.ones((NW, 1), jnp.float32)
    return lax.dot_general(parts, ones, (((0,), (0,)), ((), ())),
                           preferred_element_type=jnp.float32)


def _tc_mm1(x_ref, w1_ref, degp_ref, hn_ref, dinv_ref):
    deg = _col_sum(degp_ref[...]) + 1.0
    dinv = lax.rsqrt(deg)
    h = jnp.dot(x_ref[...], w1_ref[...], preferred_element_type=jnp.float32)
    hn_ref[...] = h * dinv
    dinv_ref[...] = dinv


def _tc_mid(accp_ref, hn_ref, dinv_ref, b1_ref, w2_ref, pn_ref):
    ssum = accp_ref[0] + accp_ref[1] + hn_ref[...]
    out1 = jnp.maximum(ssum * dinv_ref[...] + b1_ref[...][None, :], 0.0)
    q = jnp.dot(out1, w2_ref[...], preferred_element_type=jnp.float32)
    pn_ref[...] = q * dinv_ref[...]


def _tc_out(acc2p_ref, pn_ref, dinv_ref, b2_ref, out_ref):
    s2 = _col_sum(acc2p_ref[...]) + pn_ref[...]
    out_ref[...] = jax.nn.sigmoid(s2 * dinv_ref[...] + b2_ref[...])


def kernel(x, edge_index, W1, b1, W2, b2):
    src = edge_index[0].astype(jnp.int32).reshape(NW, NB, B)
    dst = edge_index[1].astype(jnp.int32).reshape(NW, NB, B)
    x_pad = jnp.zeros((NPAD, D), jnp.float32).at[:N].set(x)
    zeros_big = jnp.zeros((RPT, D), jnp.float32)
    zeros_small = jnp.zeros((NPAD,), jnp.float32)

    degp = _sc_degree(dst, zeros_small)

    hn, dinv = pl.pallas_call(
        _tc_mm1,
        grid=(_GRID,),
        in_specs=[
            pl.BlockSpec((_BR, D), lambda i: (i, 0)),
            pl.BlockSpec((D, D), lambda i: (0, 0)),
            pl.BlockSpec((NW, _BR), lambda i: (0, i)),
        ],
        out_specs=[
            pl.BlockSpec((_BR, D), lambda i: (i, 0)),
            pl.BlockSpec((_BR, 1), lambda i: (i, 0)),
        ],
        out_shape=[
            jax.ShapeDtypeStruct((NPAD, D), jnp.float32),
            jax.ShapeDtypeStruct((NPAD, 1), jnp.float32),
        ],
    )(x_pad, W1, degp)

    accp = _sc_prop_rows(src.reshape(NW, NCH, CHB, B),
                         dst.reshape(NW, NCH, CHB, B), hn, zeros_big)

    pn = pl.pallas_call(
        _tc_mid,
        grid=(_GRID,),
        in_specs=[
            pl.BlockSpec((NC, _BR, D), lambda i: (0, i, 0)),
            pl.BlockSpec((_BR, D), lambda i: (i, 0)),
            pl.BlockSpec((_BR, 1), lambda i: (i, 0)),
            pl.BlockSpec((D,), lambda i: (0,)),
            pl.BlockSpec((D, 1), lambda i: (0, 0)),
        ],
        out_specs=pl.BlockSpec((_BR, 1), lambda i: (i, 0)),
        out_shape=jax.ShapeDtypeStruct((NPAD, 1), jnp.float32),
    )(accp, hn, dinv, b1, W2)

    acc2p = _sc_prop_scalar(src, dst, pn.reshape(NPAD), zeros_small)

    out = pl.pallas_call(
        _tc_out,
        grid=(_GRID,),
        in_specs=[
            pl.BlockSpec((NW, _BR), lambda i: (0, i)),
            pl.BlockSpec((_BR, 1), lambda i: (i, 0)),
            pl.BlockSpec((_BR, 1), lambda i: (i, 0)),
            pl.BlockSpec((1,), lambda i: (0,)),
        ],
        out_specs=pl.BlockSpec((_BR, 1), lambda i: (i, 0)),
        out_shape=jax.ShapeDtypeStruct((NPAD, 1), jnp.float32),
    )(acc2p, pn, dinv, b2)

    return out[:N]


# speedup vs baseline: 1.0591x; 1.0098x over previous
import functools

import jax
import jax.numpy as jnp
from jax import lax
from jax.experimental import pallas as pl
from jax.experimental.pallas import tpu as pltpu
from jax.experimental.pallas import tpu_sc as plsc

N = 10000
E = 320000
D = 128

NC = 2
NS = 16
NW = NC * NS
EPT = E // NW
B = 80
NB = EPT // B
NPAD = 10240
RPT = NPAD // NS

_MESH = plsc.VectorSubcoreMesh(core_axis_name="c", subcore_axis_name="s")


@functools.partial(
    pl.kernel,
    out_type=jax.ShapeDtypeStruct((NW, NPAD), jnp.float32),
    mesh=_MESH,
    scratch_types=[
        pltpu.VMEM((NB, B), jnp.int32),
        pltpu.VMEM((NPAD,), jnp.float32),
    ],
    compiler_params=pltpu.CompilerParams(needs_layout_passes=False),
)
def _sc_degree(dst_hbm, zeros_hbm, out_hbm, dst_v, acc):
    c = lax.axis_index("c")
    s = lax.axis_index("s")
    wid = s * NC + c
    pltpu.sync_copy(zeros_hbm, acc)
    pltpu.sync_copy(dst_hbm.at[wid], dst_v)
    ones16 = jnp.full((16,), 1.0, jnp.float32)

    def body(m, carry):
        for r in range(B // 16):
            d16 = dst_v[m, pl.ds(r * 16, 16)]
            plsc.addupdate_scatter(acc, [d16], ones16)
        return carry

    lax.fori_loop(0, NB, body, 0)
    pltpu.sync_copy(acc, out_hbm.at[wid])


_NRING = 3


NCH = 5
CHB = NB // NCH


@functools.partial(
    pl.kernel,
    out_type=jax.ShapeDtypeStruct((NC, NPAD, D), jnp.float32),
    mesh=_MESH,
    scratch_types=(
        [pltpu.VMEM((CHB, B), jnp.int32) for _ in range(2)]
        + [pltpu.VMEM((B, D), jnp.float32) for _ in range(_NRING)]
        + [pltpu.SemaphoreType.DMA for _ in range(_NRING)]
        + [pltpu.VMEM_SHARED((NPAD, D), jnp.float32)]
    ),
)
def _sc_prop_rows(src_hbm, dst_hbm, tab_hbm, zeros_hbm, out_hbm, *refs):
    src_v, dst_v = refs[0], refs[1]
    bufs = refs[2:2 + _NRING]
    gsem = refs[2 + _NRING:2 + 2 * _NRING]
    acc = refs[2 + 2 * _NRING]
    c = lax.axis_index("c")
    s = lax.axis_index("s")
    wid = s * NC + c
    rows = pl.ds(s * RPT, RPT)

    @pl.when(c == 0)
    def _():
        pltpu.sync_copy(tab_hbm.at[rows], acc.at[rows])

    @pl.when(c != 0)
    def _():
        pltpu.sync_copy(zeros_hbm, acc.at[rows])

    plsc.subcore_barrier()

    dummy = tab_hbm.at[pl.ds(0, B)]

    def wait_g(r):
        pltpu.make_async_copy(dummy, bufs[r], gsem[r]).wait()

    def gath(j, r):
        pltpu.async_copy(tab_hbm.at[src_v.at[j]], bufs[r], gsem[r])

    def scat(j, r):
        pltpu.sync_copy(bufs[r], acc.at[dst_v.at[j]], add=True)

    for ch in range(NCH):
        pltpu.sync_copy(src_hbm.at[wid, ch], src_v)
        pltpu.sync_copy(dst_hbm.at[wid, ch], dst_v)
        gath(0, 0)
        gath(1, 1)

        def triple(m, carry):
            for r in range(3):
                j = 3 * m + r
                gath(j + 2, (r + 2) % 3)
                wait_g(r)
                scat(j, r)
            return carry

        lax.fori_loop(0, (CHB - 4) // 3, triple, 0)
        for j in range(CHB - 4, CHB):
            if j + 2 < CHB:
                gath(j + 2, (j + 2) % 3)
            wait_g(j % 3)
            scat(j, j % 3)
    plsc.subcore_barrier()
    pltpu.sync_copy(acc.at[rows], out_hbm.at[c, rows])


@functools.partial(
    pl.kernel,
    out_type=jax.ShapeDtypeStruct((NW, NPAD), jnp.float32),
    mesh=_MESH,
    scratch_types=[
        pltpu.VMEM((NB, B), jnp.int32),
        pltpu.VMEM((NB, B), jnp.int32),
        pltpu.VMEM((NPAD,), jnp.float32),
        pltpu.VMEM((NPAD,), jnp.float32),
    ],
    compiler_params=pltpu.CompilerParams(needs_layout_passes=False),
)
def _sc_prop_scalar(src_hbm, dst_hbm, tab_hbm, zeros_hbm, out_hbm,
                    src_v, dst_v, tab_v, acc):
    c = lax.axis_index("c")
    s = lax.axis_index("s")
    wid = s * NC + c
    pltpu.sync_copy(zeros_hbm, acc)
    pltpu.sync_copy(tab_hbm, tab_v)
    pltpu.sync_copy(src_hbm.at[wid], src_v)
    pltpu.sync_copy(dst_hbm.at[wid], dst_v)

    def body(m, carry):
        for r in range(B // 16):
            s16 = src_v[m, pl.ds(r * 16, 16)]
            d16 = dst_v[m, pl.ds(r * 16, 16)]
            vals = plsc.load_gather(tab_v, [s16])
            plsc.addupdate_scatter(acc, [d16], vals)
        return carry

    lax.fori_loop(0, NB, body, 0)
    pltpu.sync_copy(acc, out_hbm.at[wid])


_GRID = 20
_BR = NPAD // _GRID


def _col_sum(parts):
    ones = jnp.ones((NW, 1), jnp.float32)
    return lax.dot_general(parts, ones, (((0,), (0,)), ((), ())),
                           preferred_element_type=jnp.float32)


def _tc_mm1(x_ref, w1_ref, degp_ref, hn_ref, dinv_ref):
    deg = _col_sum(degp_ref[...]) + 1.0
    dinv = lax.rsqrt(deg)
    h = jnp.dot(x_ref[...], w1_ref[...], preferred_element_type=jnp.float32)
    hn_ref[...] = h * dinv
    dinv_ref[...] = dinv


def _tc_mid(accp_ref, dinv_ref, b1_ref, w2_ref, pn_ref):
    ssum = accp_ref[0] + accp_ref[1]
    out1 = jnp.maximum(ssum * dinv_ref[...] + b1_ref[...][None, :], 0.0)
    q = jnp.dot(out1, w2_ref[...], preferred_element_type=jnp.float32)
    pn_ref[...] = q * dinv_ref[...]


def _tc_out(acc2p_ref, pn_ref, dinv_ref, b2_ref, out_ref):
    s2 = _col_sum(acc2p_ref[...]) + pn_ref[...]
    out_ref[...] = jax.nn.sigmoid(s2 * dinv_ref[...] + b2_ref[...])


def kernel(x, edge_index, W1, b1, W2, b2):
    src = edge_index[0].astype(jnp.int32).reshape(NW, NB, B)
    dst = edge_index[1].astype(jnp.int32).reshape(NW, NB, B)
    x_pad = jnp.zeros((NPAD, D), jnp.float32).at[:N].set(x)
    zeros_big = jnp.zeros((RPT, D), jnp.float32)
    zeros_small = jnp.zeros((NPAD,), jnp.float32)

    degp = _sc_degree(dst, zeros_small)

    hn, dinv = pl.pallas_call(
        _tc_mm1,
        grid=(_GRID,),
        in_specs=[
            pl.BlockSpec((_BR, D), lambda i: (i, 0)),
            pl.BlockSpec((D, D), lambda i: (0, 0)),
            pl.BlockSpec((NW, _BR), lambda i: (0, i)),
        ],
        out_specs=[
            pl.BlockSpec((_BR, D), lambda i: (i, 0)),
            pl.BlockSpec((_BR, 1), lambda i: (i, 0)),
        ],
        out_shape=[
            jax.ShapeDtypeStruct((NPAD, D), jnp.float32),
            jax.ShapeDtypeStruct((NPAD, 1), jnp.float32),
        ],
    )(x_pad, W1, degp)

    accp = _sc_prop_rows(src.reshape(NW, NCH, CHB, B),
                         dst.reshape(NW, NCH, CHB, B), hn, zeros_big)

    pn = pl.pallas_call(
        _tc_mid,
        grid=(_GRID,),
        in_specs=[
            pl.BlockSpec((NC, _BR, D), lambda i: (0, i, 0)),
            pl.BlockSpec((_BR, 1), lambda i: (i, 0)),
            pl.BlockSpec((D,), lambda i: (0,)),
            pl.BlockSpec((D, 1), lambda i: (0, 0)),
        ],
        out_specs=pl.BlockSpec((_BR, 1), lambda i: (i, 0)),
        out_shape=jax.ShapeDtypeStruct((NPAD, 1), jnp.float32),
    )(accp, dinv, b1, W2)

    acc2p = _sc_prop_scalar(src, dst, pn.reshape(NPAD), zeros_small)

    out = pl.pallas_call(
        _tc_out,
        grid=(_GRID,),
        in_specs=[
            pl.BlockSpec((NW, _BR), lambda i: (0, i)),
            pl.BlockSpec((_BR, 1), lambda i: (i, 0)),
            pl.BlockSpec((_BR, 1), lambda i: (i, 0)),
            pl.BlockSpec((1,), lambda i: (0,)),
        ],
        out_specs=pl.BlockSpec((_BR, 1), lambda i: (i, 0)),
        out_shape=jax.ShapeDtypeStruct((NPAD, 1), jnp.float32),
    )(acc2p, pn, dinv, b2)

    return out[:N]
